# baseline (speedup 1.0000x reference)
import functools

import jax
import jax.numpy as jnp
from jax.experimental import pallas as pl
from jax.experimental.pallas import tpu as pltpu

_D = 256
_K = 512
_NVQ = 4
_T = 4096
_H = 4
_TS = _T // _H


def _rvq_kernel(x_ref, cbt_ref, hi_ref, mid_ref, lo_ref, wt_ref, b_ref,
                y_ref, idx_ref, loss_ref, p3_ref, cbsq_ref):
    step = pl.program_id(0)

    @pl.when(step == 0)
    def _precompute():
        wtb = wt_ref[...].astype(jnp.bfloat16)
        p3_ref[...] = jax.lax.dot_general(
            hi_ref[3], wtb, (((1,), (0,)), ((), ())),
            preferred_element_type=jnp.float32).astype(jnp.bfloat16)
        for s in range(_NVQ):
            cbt = cbt_ref[s]
            cbsq_ref[s:s + 1, :] = jnp.sum(cbt * cbt, axis=0, keepdims=True)

    iota = jax.lax.broadcasted_iota(jnp.int32, (_TS, _K), 1)
    dot = lambda a, b: jax.lax.dot_general(
        a, b, (((1,), (0,)), ((), ())), preferred_element_type=jnp.float32)

    xs = [x_ref[h * _TS:(h + 1) * _TS, :] for h in range(_H)]
    rs = list(xs)
    ys = [jnp.zeros((_TS, _D), jnp.float32) for _ in range(_H)]
    parts = []

    for s in range(_NVQ):
        cbt = cbt_ref[s]
        cbsq = cbsq_ref[s:s + 1, :]
        for h in range(_H):
            r = rs[h]
            rsq = jnp.sum(r * r, axis=1, keepdims=True)
            sc = dot(r, cbt)
            d = (rsq - 2.0 * sc) + cbsq
            dmin = jnp.min(d, axis=1, keepdims=True)
            idx = jnp.min(jnp.where(d == dmin, iota, _K),
                          axis=1, keepdims=True)
            onehot = (iota == idx).astype(jnp.bfloat16)
            if s == 3:
                out = (dot(ys[h], wt_ref[...])
                       + dot(onehot, p3_ref[...]) + b_ref[...])
                y_ref[h * _TS:(h + 1) * _TS, :] = out
                continue
            q = ((dot(onehot, hi_ref[s]) + dot(onehot, mid_ref[s]))
                 + dot(onehot, lo_ref[s]))
            if s == 0:
                idx_ref[h * _TS:(h + 1) * _TS, :] = idx
                parts.append(jnp.sum((xs[h] - q) ** 2))
            x_hat = r + (q - r)
            rs[h] = r - x_hat
            ys[h] = ys[h] + x_hat

    part = sum(parts).reshape(1, 1)

    @pl.when(step == 0)
    def _init():
        loss_ref[...] = part

    @pl.when(step != 0)
    def _acc():
        loss_ref[...] += part


@functools.partial(jax.jit, static_argnames=())
def kernel(x, codebooks, W, b):
    n = x.shape[0]
    x = x.reshape(n, _D)
    grid = n // _T

    cbt = jnp.transpose(codebooks, (0, 2, 1))
    mask = jnp.uint32(0xFFFF0000)
    bits = jax.lax.bitcast_convert_type(codebooks, jnp.uint32)
    hi32 = jax.lax.bitcast_convert_type(bits & mask, jnp.float32)
    r1 = codebooks - hi32
    r1bits = jax.lax.bitcast_convert_type(r1, jnp.uint32)
    mid32 = jax.lax.bitcast_convert_type(r1bits & mask, jnp.float32)
    hi = hi32.astype(jnp.bfloat16)
    mid = mid32.astype(jnp.bfloat16)
    lo = (r1 - mid32).astype(jnp.bfloat16)
    wt = W.T
    b2 = b.reshape(1, _D)

    full = lambda *_: tuple(0 for _ in range(3))
    y, idx, loss_sum = pl.pallas_call(
        _rvq_kernel,
        grid=(grid,),
        in_specs=[
            pl.BlockSpec((_T, _D), lambda i: (i, 0)),
            pl.BlockSpec((_NVQ, _D, _K), full),
            pl.BlockSpec((_NVQ, _K, _D), full),
            pl.BlockSpec((_NVQ, _K, _D), full),
            pl.BlockSpec((_NVQ, _K, _D), full),
            pl.BlockSpec((_D, _D), lambda i: (0, 0)),
            pl.BlockSpec((1, _D), lambda i: (0, 0)),
        ],
        out_specs=[
            pl.BlockSpec((_T, _D), lambda i: (i, 0)),
            pl.BlockSpec((_T, 1), lambda i: (i, 0)),
            pl.BlockSpec((1, 1), lambda i: (0, 0)),
        ],
        out_shape=[
            jax.ShapeDtypeStruct((n, _D), jnp.float32),
            jax.ShapeDtypeStruct((n, 1), jnp.int32),
            jax.ShapeDtypeStruct((1, 1), jnp.float32),
        ],
        scratch_shapes=[pltpu.VMEM((_K, _D), jnp.bfloat16),
                        pltpu.VMEM((8, _K), jnp.float32)],
        compiler_params=pltpu.CompilerParams(
            dimension_semantics=("arbitrary",)),
    )(x, cbt, hi, mid, lo, wt, b2)

    y = y.reshape(n, 1, _D)
    idx = idx.reshape(n)
    commit = (loss_sum / (n * _D)).reshape(())
    return y, idx, commit


# speedup vs baseline: 1.0054x; 1.0054x over previous
import functools

import jax
import jax.numpy as jnp
from jax.experimental import pallas as pl
from jax.experimental.pallas import tpu as pltpu

_D = 256
_K = 512
_NVQ = 4
_T = 2048
_H = 2
_TS = _T // _H


def _rvq_kernel(x_ref, cbt_ref, hi_ref, mid_ref, lo_ref, wt_ref, b_ref,
                y_ref, idx_ref, loss_ref, p3_ref, cbsq_ref):
    step = pl.program_id(0)

    @pl.when(step == 0)
    def _precompute():
        wtb = wt_ref[...].astype(jnp.bfloat16)
        p3_ref[...] = jax.lax.dot_general(
            hi_ref[3], wtb, (((1,), (0,)), ((), ())),
            preferred_element_type=jnp.float32).astype(jnp.bfloat16)
        for s in range(_NVQ):
            cbt = cbt_ref[s]
            cbsq_ref[s:s + 1, :] = jnp.sum(cbt * cbt, axis=0, keepdims=True)

    iota = jax.lax.broadcasted_iota(jnp.int32, (_TS, _K), 1)
    dot = lambda a, b: jax.lax.dot_general(
        a, b, (((1,), (0,)), ((), ())), preferred_element_type=jnp.float32)

    xs = [x_ref[h * _TS:(h + 1) * _TS, :] for h in range(_H)]
    rs = list(xs)
    ys = [jnp.zeros((_TS, _D), jnp.float32) for _ in range(_H)]
    parts = []

    for s in range(_NVQ):
        cbt = cbt_ref[s]
        cbsq = cbsq_ref[s:s + 1, :]
        for h in range(_H):
            r = rs[h]
            rsq = jnp.sum(r * r, axis=1, keepdims=True)
            sc = dot(r, cbt)
            d = (rsq - 2.0 * sc) + cbsq
            dmin = jnp.min(d, axis=1, keepdims=True)
            idx = jnp.min(jnp.where(d == dmin, iota, _K),
                          axis=1, keepdims=True)
            onehot = (iota == idx).astype(jnp.bfloat16)
            if s == 3:
                out = (dot(ys[h], wt_ref[...])
                       + dot(onehot, p3_ref[...]) + b_ref[...])
                y_ref[h * _TS:(h + 1) * _TS, :] = out
                continue
            q = ((dot(onehot, hi_ref[s]) + dot(onehot, mid_ref[s]))
                 + dot(onehot, lo_ref[s]))
            if s == 0:
                idx_ref[h * _TS:(h + 1) * _TS, :] = idx
                parts.append(jnp.sum((xs[h] - q) ** 2))
            x_hat = r + (q - r)
            rs[h] = r - x_hat
            ys[h] = ys[h] + x_hat

    part = sum(parts).reshape(1, 1)

    @pl.when(step == 0)
    def _init():
        loss_ref[...] = part

    @pl.when(step != 0)
    def _acc():
        loss_ref[...] += part


@functools.partial(jax.jit, static_argnames=())
def kernel(x, codebooks, W, b):
    n = x.shape[0]
    x = x.reshape(n, _D)
    grid = n // _T

    cbt = jnp.transpose(codebooks, (0, 2, 1))
    mask = jnp.uint32(0xFFFF0000)
    bits = jax.lax.bitcast_convert_type(codebooks, jnp.uint32)
    hi32 = jax.lax.bitcast_convert_type(bits & mask, jnp.float32)
    r1 = codebooks - hi32
    r1bits = jax.lax.bitcast_convert_type(r1, jnp.uint32)
    mid32 = jax.lax.bitcast_convert_type(r1bits & mask, jnp.float32)
    hi = hi32.astype(jnp.bfloat16)
    mid = mid32.astype(jnp.bfloat16)
    lo = (r1 - mid32).astype(jnp.bfloat16)
    wt = W.T
    b2 = b.reshape(1, _D)

    full = lambda *_: tuple(0 for _ in range(3))
    y, idx, loss_sum = pl.pallas_call(
        _rvq_kernel,
        grid=(grid,),
        in_specs=[
            pl.BlockSpec((_T, _D), lambda i: (i, 0)),
            pl.BlockSpec((_NVQ, _D, _K), full),
            pl.BlockSpec((_NVQ, _K, _D), full),
            pl.BlockSpec((_NVQ, _K, _D), full),
            pl.BlockSpec((_NVQ, _K, _D), full),
            pl.BlockSpec((_D, _D), lambda i: (0, 0)),
            pl.BlockSpec((1, _D), lambda i: (0, 0)),
        ],
        out_specs=[
            pl.BlockSpec((_T, _D), lambda i: (i, 0)),
            pl.BlockSpec((_T, 1), lambda i: (i, 0)),
            pl.BlockSpec((1, 1), lambda i: (0, 0)),
        ],
        out_shape=[
            jax.ShapeDtypeStruct((n, _D), jnp.float32),
            jax.ShapeDtypeStruct((n, 1), jnp.int32),
            jax.ShapeDtypeStruct((1, 1), jnp.float32),
        ],
        scratch_shapes=[pltpu.VMEM((_K, _D), jnp.bfloat16),
                        pltpu.VMEM((8, _K), jnp.float32)],
        compiler_params=pltpu.CompilerParams(
            dimension_semantics=("arbitrary",)),
    )(x, cbt, hi, mid, lo, wt, b2)

    y = y.reshape(n, 1, _D)
    idx = idx.reshape(n)
    commit = (loss_sum / (n * _D)).reshape(())
    return y, idx, commit


# speedup vs baseline: 1.0350x; 1.0295x over previous
import functools

import jax
import jax.numpy as jnp
from jax.experimental import pallas as pl
from jax.experimental.pallas import tpu as pltpu

_D = 256
_K = 512
_NVQ = 4
_T = 2048
_H = 2
_TS = _T // _H


def _rvq_kernel(x_ref, cb_ref, w_ref, b_ref,
                y_ref, idx_ref, loss_ref,
                cbt_ref, hi_ref, mid_ref, lo_ref, wt_ref, p3_ref, cbsq_ref):
    step = pl.program_id(0)

    @pl.when(step == 0)
    def _precompute():
        mask = jnp.uint32(0xFFFF0000)
        wt = w_ref[...].T
        wt_ref[...] = wt
        for s in range(_NVQ):
            cb = cb_ref[s]
            cbt = cb.T
            cbt_ref[s] = cbt
            cbsq_ref[s:s + 1, :] = jnp.sum(cbt * cbt, axis=0, keepdims=True)
            bits = jax.lax.bitcast_convert_type(cb, jnp.uint32)
            hi32 = jax.lax.bitcast_convert_type(bits & mask, jnp.float32)
            r1 = cb - hi32
            r1bits = jax.lax.bitcast_convert_type(r1, jnp.uint32)
            mid32 = jax.lax.bitcast_convert_type(r1bits & mask, jnp.float32)
            hi_ref[s] = hi32.astype(jnp.bfloat16)
            mid_ref[s] = mid32.astype(jnp.bfloat16)
            lo_ref[s] = (r1 - mid32).astype(jnp.bfloat16)
            if s == 3:
                p3_ref[...] = jax.lax.dot_general(
                    hi32.astype(jnp.bfloat16), wt.astype(jnp.bfloat16),
                    (((1,), (0,)), ((), ())),
                    preferred_element_type=jnp.float32).astype(jnp.bfloat16)

    iota = jax.lax.broadcasted_iota(jnp.int32, (_TS, _K), 1)
    dot = lambda a, b: jax.lax.dot_general(
        a, b, (((1,), (0,)), ((), ())), preferred_element_type=jnp.float32)

    xs = [x_ref[h * _TS:(h + 1) * _TS, :] for h in range(_H)]
    rs = list(xs)
    ys = [jnp.zeros((_TS, _D), jnp.float32) for _ in range(_H)]
    parts = []

    for s in range(_NVQ):
        cbt = cbt_ref[s]
        cbsq = cbsq_ref[s:s + 1, :]
        for h in range(_H):
            r = rs[h]
            rsq = jnp.sum(r * r, axis=1, keepdims=True)
            sc = dot(r, cbt)
            d = (rsq - 2.0 * sc) + cbsq
            dmin = jnp.min(d, axis=1, keepdims=True)
            idx = jnp.min(jnp.where(d == dmin, iota, _K),
                          axis=1, keepdims=True)
            onehot = (iota == idx).astype(jnp.bfloat16)
            if s == 3:
                out = (dot(ys[h], wt_ref[...])
                       + dot(onehot, p3_ref[...]) + b_ref[...])
                y_ref[h * _TS:(h + 1) * _TS, :] = out
                continue
            q = ((dot(onehot, hi_ref[s]) + dot(onehot, mid_ref[s]))
                 + dot(onehot, lo_ref[s]))
            if s == 0:
                idx_ref[h * _TS:(h + 1) * _TS, :] = idx
                parts.append(jnp.sum((xs[h] - q) ** 2))
            x_hat = r + (q - r)
            rs[h] = r - x_hat
            ys[h] = ys[h] + x_hat

    part = sum(parts).reshape(1, 1)

    @pl.when(step == 0)
    def _init():
        loss_ref[...] = part

    @pl.when(step != 0)
    def _acc():
        loss_ref[...] += part


@functools.partial(jax.jit, static_argnames=())
def kernel(x, codebooks, W, b):
    n = x.shape[0]
    x = x.reshape(n, _D)
    grid = n // _T
    b2 = b.reshape(1, _D)

    y, idx, loss_sum = pl.pallas_call(
        _rvq_kernel,
        grid=(grid,),
        in_specs=[
            pl.BlockSpec((_T, _D), lambda i: (i, 0)),
            pl.BlockSpec((_NVQ, _K, _D), lambda i: (0, 0, 0)),
            pl.BlockSpec((_D, _D), lambda i: (0, 0)),
            pl.BlockSpec((1, _D), lambda i: (0, 0)),
        ],
        out_specs=[
            pl.BlockSpec((_T, _D), lambda i: (i, 0)),
            pl.BlockSpec((_T, 1), lambda i: (i, 0)),
            pl.BlockSpec((1, 1), lambda i: (0, 0)),
        ],
        out_shape=[
            jax.ShapeDtypeStruct((n, _D), jnp.float32),
            jax.ShapeDtypeStruct((n, 1), jnp.int32),
            jax.ShapeDtypeStruct((1, 1), jnp.float32),
        ],
        scratch_shapes=[
            pltpu.VMEM((_NVQ, _D, _K), jnp.float32),
            pltpu.VMEM((_NVQ, _K, _D), jnp.bfloat16),
            pltpu.VMEM((_NVQ, _K, _D), jnp.bfloat16),
            pltpu.VMEM((_NVQ, _K, _D), jnp.bfloat16),
            pltpu.VMEM((_D, _D), jnp.float32),
            pltpu.VMEM((_K, _D), jnp.bfloat16),
            pltpu.VMEM((8, _K), jnp.float32),
        ],
        compiler_params=pltpu.CompilerParams(
            dimension_semantics=("arbitrary",)),
    )(x, codebooks, W, b2)

    y = y.reshape(n, 1, _D)
    idx = idx.reshape(n)
    commit = (loss_sum / (n * _D)).reshape(())
    return y, idx, commit
